# baseline (speedup 1.0000x reference)
import functools

import jax
import jax.numpy as jnp
from jax import lax
from jax.experimental import pallas as pl
from jax.experimental.pallas import tpu as pltpu
from jax.experimental.pallas import tpu_sc as plsc

_N = 320000
_D = 128
_DE = 16
_NW = 32
_RPW = _N // _NW
_CH = 400

_SB = 100


def _gate_body(e_hbm, gate_hbm, ebuf0, ebuf1, gbuf0, gbuf1, isem, osem):
    ebuf = (ebuf0, ebuf1)
    gbuf = (gbuf0, gbuf1)
    wid = lax.axis_index("s") * 2 + lax.axis_index("c")
    base = wid * _RPW
    lanes = lax.iota(jnp.int32, _DE)
    colv = [jnp.full((_DE,), j, jnp.int32) for j in range(_DE)]
    nch = _RPW // _CH

    def row0(c):
        return pl.multiple_of(base + c * _CH, 8)

    def in_copy(c):
        return pltpu.make_async_copy(
            e_hbm.at[pl.ds(row0(c), _CH)], ebuf[c % 2], isem.at[c % 2])

    def out_copy(c):
        return pltpu.make_async_copy(
            gbuf[c % 2], gate_hbm.at[pl.ds(row0(c), _CH)], osem.at[c % 2])

    def compute(s):
        def block5(p, carry2):
            r0 = p * (5 * _DE)
            for q in range(5):
                rows = r0 + q * _DE + lanes
                g = [plsc.load_gather(ebuf[s], [rows, colv[j]])
                     for j in range(_DE)]
                while len(g) > 1:
                    g = [g[k] + g[k + 1] for k in range(0, len(g), 2)]
                gbuf[s][pl.ds(r0 + q * _DE, _DE)] = (
                    1.0 / (1.0 + jnp.exp(g[0] * (-1.0 / _DE))))
            return carry2

        lax.fori_loop(0, _CH // (5 * _DE), block5, 0)

    in_copy(0).start()
    for c in range(nch):
        s = c % 2
        if c + 1 < nch:
            in_copy(c + 1).start()
        in_copy(c).wait()
        if c >= 2:
            out_copy(c - 2).wait()
        compute(s)
        out_copy(c).start()
    if nch >= 2:
        out_copy(nch - 2).wait()
    out_copy(nch - 1).wait()


@functools.partial(jax.jit, static_argnames=())
def _gate_sc(e):
    mesh = plsc.VectorSubcoreMesh(core_axis_name="c", subcore_axis_name="s")
    return pl.kernel(
        _gate_body,
        out_type=jax.ShapeDtypeStruct((_N,), jnp.float32),
        mesh=mesh,
        scratch_types=[
            pltpu.VMEM((_CH, _DE), jnp.float32),
            pltpu.VMEM((_CH, _DE), jnp.float32),
            pltpu.VMEM((_CH,), jnp.float32),
            pltpu.VMEM((_CH,), jnp.float32),
            pltpu.SemaphoreType.DMA((2,)),
            pltpu.SemaphoreType.DMA((2,)),
        ],
        compiler_params=pltpu.CompilerParams(
            use_tc_tiling_on_sc=True, needs_layout_passes=False),
    )(e)


def _fma_body(src_ref, g_ref, tgt_ref, out_ref):
    g3 = g_ref[...][:, :, :, None]
    out_ref[...] = src_ref[...] * g3 + tgt_ref[...]


def kernel(src, e, tgt):
    n, d = src.shape
    ns = n // d
    ng = ns // _SB
    gate = _gate_sc(e)
    g3 = jnp.zeros((ng, _SB, d), jnp.float32) + gate[0] * 0.0
    src4 = src.reshape(ng, _SB, d, d)
    tgt4 = tgt.reshape(ng, _SB, d, d)
    out4 = pl.pallas_call(
        _fma_body,
        grid=(ng,),
        in_specs=[
            pl.BlockSpec((1, _SB, d, d), lambda i: (i, 0, 0, 0)),
            pl.BlockSpec((1, _SB, d), lambda i: (i, 0, 0)),
            pl.BlockSpec((1, _SB, d, d), lambda i: (i, 0, 0, 0)),
        ],
        out_specs=pl.BlockSpec((1, _SB, d, d), lambda i: (i, 0, 0, 0)),
        out_shape=jax.ShapeDtypeStruct((ng, _SB, d, d), src.dtype),
        compiler_params=pltpu.CompilerParams(
            dimension_semantics=("parallel",),
        ),
    )(src4, g3, tgt4)
    return out4.reshape(n, d)


# speedup vs baseline: 1.0317x; 1.0317x over previous
import functools

import jax
import jax.numpy as jnp
from jax import lax
from jax.experimental import pallas as pl
from jax.experimental.pallas import tpu as pltpu
from jax.experimental.pallas import tpu_sc as plsc

_N = 320000
_D = 128
_DE = 16
_NW = 32
_C = 5
_NC = _N // _C
_RPW = _NC // _NW
_CH = 400

_SB = 50
_NGC = _NC // (_SB * _D)


def _gate_body(c0, e_hbm, gate_hbm, ebuf0, ebuf1, gbuf0, gbuf1, isem, osem):
    ebuf = (ebuf0, ebuf1)
    gbuf = (gbuf0, gbuf1)
    wid = lax.axis_index("s") * 2 + lax.axis_index("c")
    base = c0 + wid * _RPW
    lanes = lax.iota(jnp.int32, _DE)
    colv = [jnp.full((_DE,), j, jnp.int32) for j in range(_DE)]
    nch = _RPW // _CH

    def row0(c):
        return pl.multiple_of(base + c * _CH, 8)

    def in_copy(c):
        return pltpu.make_async_copy(
            e_hbm.at[pl.ds(row0(c), _CH)], ebuf[c % 2], isem.at[c % 2])

    def out_copy(c):
        return pltpu.make_async_copy(
            gbuf[c % 2],
            gate_hbm.at[pl.ds(pl.multiple_of(wid * _RPW + c * _CH, 8), _CH)],
            osem.at[c % 2])

    def compute(s):
        def block5(p, carry2):
            r0 = p * (5 * _DE)
            for q in range(5):
                rows = r0 + q * _DE + lanes
                g = [plsc.load_gather(ebuf[s], [rows, colv[j]])
                     for j in range(_DE)]
                while len(g) > 1:
                    g = [g[k] + g[k + 1] for k in range(0, len(g), 2)]
                gbuf[s][pl.ds(r0 + q * _DE, _DE)] = (
                    1.0 / (1.0 + jnp.exp(g[0] * (-1.0 / _DE))))
            return carry2

        lax.fori_loop(0, _CH // (5 * _DE), block5, 0)

    in_copy(0).start()
    for c in range(nch):
        s = c % 2
        if c + 1 < nch:
            in_copy(c + 1).start()
        in_copy(c).wait()
        if c >= 2:
            out_copy(c - 2).wait()
        compute(s)
        out_copy(c).start()
    if nch >= 2:
        out_copy(nch - 2).wait()
    out_copy(nch - 1).wait()


def _gate_sc(e, cidx):
    mesh = plsc.VectorSubcoreMesh(core_axis_name="c", subcore_axis_name="s")
    return pl.kernel(
        functools.partial(_gate_body, cidx * _NC),
        out_type=jax.ShapeDtypeStruct((_NC,), jnp.float32),
        mesh=mesh,
        scratch_types=[
            pltpu.VMEM((_CH, _DE), jnp.float32),
            pltpu.VMEM((_CH, _DE), jnp.float32),
            pltpu.VMEM((_CH,), jnp.float32),
            pltpu.VMEM((_CH,), jnp.float32),
            pltpu.SemaphoreType.DMA((2,)),
            pltpu.SemaphoreType.DMA((2,)),
        ],
        compiler_params=pltpu.CompilerParams(
            use_tc_tiling_on_sc=True, needs_layout_passes=False),
    )(e)


def _fma_first_body(src_ref, g_ref, tgt_ref, out_ref):
    g3 = g_ref[...][:, :, :, None]
    out_ref[...] = src_ref[...] * g3 + tgt_ref[...]


def _fma_next_body(prev_ref, src_ref, g_ref, tgt_ref, out_ref):
    del prev_ref
    g3 = g_ref[...][:, :, :, None]
    out_ref[...] = src_ref[...] * g3 + tgt_ref[...]


def _fma_chunk(cidx, src4, g3, tgt4, prev):
    ns = _N // _D
    ng = ns // _SB

    def blk4(i):
        return (i + cidx * _NGC, 0, 0, 0)

    def blk3(i):
        return (i, 0, 0)

    spec4 = pl.BlockSpec((1, _SB, _D, _D), blk4)
    spec3 = pl.BlockSpec((1, _SB, _D), blk3)
    out_shape = jax.ShapeDtypeStruct((ng, _SB, _D, _D), jnp.float32)
    if prev is None:
        return pl.pallas_call(
            _fma_first_body,
            grid=(_NGC,),
            in_specs=[spec4, spec3, spec4],
            out_specs=pl.BlockSpec((1, _SB, _D, _D), blk4),
            out_shape=out_shape,
            compiler_params=pltpu.CompilerParams(
                dimension_semantics=("parallel",),
            ),
        )(src4, g3, tgt4)
    return pl.pallas_call(
        _fma_next_body,
        grid=(_NGC,),
        in_specs=[
            pl.BlockSpec(memory_space=pl.ANY),
            spec4, spec3, spec4,
        ],
        out_specs=pl.BlockSpec((1, _SB, _D, _D), blk4),
        out_shape=out_shape,
        input_output_aliases={0: 0},
        compiler_params=pltpu.CompilerParams(
            dimension_semantics=("parallel",),
        ),
    )(prev, src4, g3, tgt4)


def kernel(src, e, tgt):
    n, d = src.shape
    ns = n // d
    ng = ns // _SB
    src4 = src.reshape(ng, _SB, d, d)
    tgt4 = tgt.reshape(ng, _SB, d, d)
    gates = [_gate_sc(e, c) for c in range(_C)]
    out = None
    for c in range(_C):
        g3 = gates[c].reshape(_NGC, _SB, d)
        out = _fma_chunk(c, src4, g3, tgt4, out)
    return out.reshape(n, d)
